# baseline (speedup 1.0000x reference)
import functools

import jax
import jax.numpy as jnp
from jax import lax
from jax.experimental import pallas as pl
from jax.experimental.pallas import tpu as pltpu
from jax.experimental.pallas import tpu_sc as plsc

_LANES = 16
_NC = 2
_NS = 16


@functools.lru_cache(maxsize=None)
def _make_sc_add(B, S, D):
    NW = _NC * _NS
    total_rows = B * S
    rows_per_w = total_rows // NW
    CHUNK = 32
    n_chunks = rows_per_w // CHUNK
    W = CHUNK * D
    UNROLL = 8

    mesh = plsc.VectorSubcoreMesh(core_axis_name="c", subcore_axis_name="s")

    @functools.partial(
        pl.kernel,
        out_type=jax.ShapeDtypeStruct((total_rows * D,), jnp.float32),
        mesh=mesh,
        scratch_types=[
            pltpu.VMEM((W,), jnp.float32),
            pltpu.VMEM((W,), jnp.float32),
        ],
    )
    def k(x_hbm, pos_hbm, out_hbm, xb, pb):
        c = lax.axis_index("c")
        s = lax.axis_index("s")
        wid = s * _NC + c
        base_row = wid * rows_per_w
        s_base = lax.rem(base_row, S)

        def chunk_body(g, carry):
            off = pl.multiple_of((base_row + g * CHUNK) * D, 8)
            poff = pl.multiple_of((s_base + g * CHUNK) * D, 8)
            pltpu.sync_copy(x_hbm.at[pl.ds(off, W)], xb)
            pltpu.sync_copy(pos_hbm.at[pl.ds(poff, W)], pb)

            def add_body(i, carry2):
                base = i * (_LANES * UNROLL)
                for u in range(UNROLL):
                    sl = pl.ds(base + u * _LANES, _LANES)
                    xb[sl] = xb[sl] + pb[sl]
                return carry2

            lax.fori_loop(0, W // (_LANES * UNROLL), add_body, 0)
            pltpu.sync_copy(xb, out_hbm.at[pl.ds(off, W)])
            return carry

        lax.fori_loop(0, n_chunks, chunk_body, 0)

    return k


def kernel(x, pos_emb):
    B, S, D = x.shape
    k = _make_sc_add(B, S, D)
    out_flat = k(x.reshape(-1), pos_emb.reshape(-1))
    return out_flat.reshape(B, S, D)


# speedup vs baseline: 1.3638x; 1.3638x over previous
import functools

import jax
import jax.numpy as jnp
from jax import lax
from jax.experimental import pallas as pl
from jax.experimental.pallas import tpu as pltpu
from jax.experimental.pallas import tpu_sc as plsc

_LANES = 16
_NC = 2
_NS = 16


@functools.lru_cache(maxsize=None)
def _make_sc_add(B, S, D):
    NW = _NC * _NS
    s_per_w = S // NW
    SCHUNK = 16
    n_iter = s_per_w // SCHUNK
    L = SCHUNK * D
    XW = B * L
    UNROLL = 4

    mesh = plsc.VectorSubcoreMesh(core_axis_name="c", subcore_axis_name="s")

    @functools.partial(
        pl.kernel,
        out_type=jax.ShapeDtypeStruct((B * S * D,), jnp.float32),
        mesh=mesh,
        scratch_types=[
            pltpu.VMEM((XW,), jnp.float32),
            pltpu.VMEM((XW,), jnp.float32),
            pltpu.VMEM((L,), jnp.float32),
            pltpu.VMEM((L,), jnp.float32),
            pltpu.SemaphoreType.DMA,
            pltpu.SemaphoreType.DMA,
            pltpu.SemaphoreType.DMA,
            pltpu.SemaphoreType.DMA,
            pltpu.SemaphoreType.DMA,
            pltpu.SemaphoreType.DMA,
        ],
    )
    def k(x_hbm, pos_hbm, out_hbm, xb0, xb1, pb0, pb1,
          sem_x0, sem_x1, sem_p0, sem_p1, sem_s0, sem_s1):
        c = lax.axis_index("c")
        s = lax.axis_index("s")
        wid = s * _NC + c
        s_base = wid * s_per_w

        xbs = (xb0, xb1)
        pbs = (pb0, pb1)
        sem_x = (sem_x0, sem_x1)
        sem_p = (sem_p0, sem_p1)
        sem_s = (sem_s0, sem_s1)
        load_h = {}
        store_h = {}

        def issue_loads(it):
            slot = it % 2
            s0 = s_base + it * SCHUNK
            poff = pl.multiple_of(s0 * D, 8)
            hp = pltpu.async_copy(pos_hbm.at[pl.ds(poff, L)],
                                  pbs[slot], sem_p[slot])
            hx = []
            for b in range(B):
                xoff = pl.multiple_of((b * S + s0) * D, 8)
                hx.append(pltpu.async_copy(
                    x_hbm.at[pl.ds(xoff, L)],
                    xbs[slot].at[pl.ds(b * L, L)], sem_x[slot]))
            load_h[it] = (hp, hx)

        def wait_loads(it):
            hp, hx = load_h.pop(it)
            hp.wait()
            for h in hx:
                h.wait()

        def compute(it):
            slot = it % 2
            xb = xbs[slot]
            pb = pbs[slot]

            def body(i, _):
                base = i * (_LANES * UNROLL)
                for u in range(UNROLL):
                    j = base + u * _LANES
                    v = pb[pl.ds(j, _LANES)]
                    for b in range(B):
                        plsc.addupdate(xb.at[pl.ds(b * L + j, _LANES)], v)
                return 0

            lax.fori_loop(0, L // (_LANES * UNROLL), body, 0)

        def issue_store(it):
            slot = it % 2
            s0 = s_base + it * SCHUNK
            hs = []
            for b in range(B):
                ooff = pl.multiple_of((b * S + s0) * D, 8)
                hs.append(pltpu.async_copy(
                    xbs[slot].at[pl.ds(b * L, L)],
                    out_hbm.at[pl.ds(ooff, L)], sem_s[slot]))
            store_h[it] = hs

        def wait_store(it):
            for h in store_h.pop(it):
                h.wait()

        issue_loads(0)
        for it in range(n_iter):
            if it + 1 < n_iter:
                if it >= 1:
                    wait_store(it - 1)
                issue_loads(it + 1)
            wait_loads(it)
            compute(it)
            issue_store(it)
        wait_store(n_iter - 2)
        wait_store(n_iter - 1)

    return k


def kernel(x, pos_emb):
    B, S, D = x.shape
    k = _make_sc_add(B, S, D)
    out_flat = k(x.reshape(-1), pos_emb.reshape(-1))
    return out_flat.reshape(B, S, D)


# speedup vs baseline: 3.9232x; 2.8767x over previous
import functools

import jax
import jax.numpy as jnp
from jax import lax
from jax.experimental import pallas as pl
from jax.experimental.pallas import tpu as pltpu
from jax.experimental.pallas import tpu_sc as plsc

_LANES = 16
_NC = 2
_NS = 16


@functools.lru_cache(maxsize=None)
def _make_sc_add(B, S, D):
    NW = _NC * _NS
    s_per_w = S // NW
    SCHUNK = 16
    n_iter = s_per_w // SCHUNK

    mesh = plsc.VectorSubcoreMesh(core_axis_name="c", subcore_axis_name="s")

    xb_types = [pltpu.VMEM((SCHUNK, D), jnp.float32)
                for _ in range(2 * B)]
    pb_types = [pltpu.VMEM((SCHUNK, D), jnp.float32) for _ in range(2)]

    @functools.partial(
        pl.kernel,
        out_type=jax.ShapeDtypeStruct((B, S, D), jnp.float32),
        mesh=mesh,
        scratch_types=xb_types + pb_types + [
            pltpu.SemaphoreType.DMA,
            pltpu.SemaphoreType.DMA,
            pltpu.SemaphoreType.DMA,
            pltpu.SemaphoreType.DMA,
            pltpu.SemaphoreType.DMA,
            pltpu.SemaphoreType.DMA,
        ],
    )
    def k(x_hbm, pos_hbm, out_hbm, *refs):
        xbs = tuple(tuple(refs[sl * B + b] for b in range(B)) for sl in range(2))
        pbs = refs[2 * B:2 * B + 2]
        sem_x = refs[2 * B + 2:2 * B + 4]
        sem_p = refs[2 * B + 4:2 * B + 6]
        sem_s = refs[2 * B + 6:2 * B + 8]

        c = lax.axis_index("c")
        s = lax.axis_index("s")
        wid = s * _NC + c
        s_base = wid * s_per_w

        load_h = {}
        store_h = {}

        def issue_loads(it):
            slot = it % 2
            s0 = pl.multiple_of(s_base + it * SCHUNK, SCHUNK)
            hp = pltpu.async_copy(pos_hbm.at[pl.ds(s0, SCHUNK)],
                                  pbs[slot], sem_p[slot])
            hx = [pltpu.async_copy(x_hbm.at[b, pl.ds(s0, SCHUNK)],
                                   xbs[slot][b], sem_x[slot])
                  for b in range(B)]
            load_h[it] = (hp, hx)

        def wait_loads(it):
            hp, hx = load_h.pop(it)
            hp.wait()
            for h in hx:
                h.wait()

        def compute(it):
            slot = it % 2
            xb = xbs[slot]
            pb = pbs[slot]

            def body(r, _):
                for j in range(D // _LANES):
                    sl = pl.ds(j * _LANES, _LANES)
                    v = pb[r, sl]
                    for b in range(B):
                        plsc.addupdate(xb[b].at[r, sl], v)
                return 0

            lax.fori_loop(0, SCHUNK, body, 0)

        def issue_store(it):
            slot = it % 2
            s0 = pl.multiple_of(s_base + it * SCHUNK, SCHUNK)
            store_h[it] = [pltpu.async_copy(xbs[slot][b],
                                            out_hbm.at[b, pl.ds(s0, SCHUNK)],
                                            sem_s[slot])
                           for b in range(B)]

        def wait_store(it):
            for h in store_h.pop(it):
                h.wait()

        issue_loads(0)
        for it in range(n_iter):
            if it + 1 < n_iter:
                if it >= 1:
                    wait_store(it - 1)
                issue_loads(it + 1)
            wait_loads(it)
            compute(it)
            issue_store(it)
        wait_store(n_iter - 2)
        wait_store(n_iter - 1)

    return k


def kernel(x, pos_emb):
    B, S, D = x.shape
    k = _make_sc_add(B, S, D)
    return k(x, pos_emb)
